# baseline (speedup 1.0000x reference)
import functools

import numpy as np
import jax
import jax.numpy as jnp
from jax import lax
from jax.experimental import pallas as pl
from jax.experimental.pallas import tpu as pltpu
from jax.experimental.pallas import tpu_sc as plsc

B, L = 4096, 200
EMBED, TOPICS = 64, 64
D = EMBED + TOPICS
N_VOCAB, N_TOPIC_VOCAB = 100000, 1000
HIDDEN, NUM_CLASSES = 256, 10

NC, NS = 2, 16
NW = NC * NS
B_PER_W = B // NW
IB = 16
NBLK = B_PER_W // IB
NSLOT = 4
LOOKAHEAD = 3
TOPIC_WORDS = TOPICS // 2

CHUNKS = ((0, 128), (128, 72))

_PERM = np.concatenate(
    [np.concatenate([np.arange(g, g + 32, 2), np.arange(g + 1, g + 32, 2)])
     for g in range(0, D, 32)])


def _pooled_sc_kernel(word_ids_hbm, topic_ids_hbm, emb_word_hbm,
                      emb_topics_packed_hbm, out_hbm,
                      idxw, idxt, wr0, wr1, wr2, wr3, topicv,
                      outv, sem0, sem1, sem2, sem3):
    wid = lax.axis_index("s") * NC + lax.axis_index("c")
    base = wid * B_PER_W
    wr = (wr0, wr1, wr2, wr3)
    sems = (sem0, sem1, sem2, sem3)

    pltpu.sync_copy(emb_topics_packed_hbm, topicv)

    iotas = tuple(lax.iota(jnp.int32, 16) + 16 * j for j in range(2))
    _splat_dnums = lax.GatherDimensionNumbers(
        offset_dims=(), collapsed_slice_dims=(0,), start_index_map=(0,))

    def lane_splat(v, i):
        idx = jnp.zeros((16, 1), jnp.int32) + i
        return lax.gather(v, idx, _splat_dnums, (1,),
                          mode=lax.GatherScatterMode.PROMISE_IN_BOUNDS)

    def issue(r, slot):
        copies = []
        for off, n in CHUNKS:
            copies.append(pltpu.async_copy(
                emb_word_hbm.at[idxw.at[r, pl.ds(off, n)]],
                wr[slot].at[pl.ds(off, n)], sems[slot]))
        return copies

    def reduce_row(slot, r_out):
        def lane_body(off):
            def inner(i, acc2):
                l = off + i
                a = list(acc2[:8])
                tids = acc2[8]
                for j in range(2):
                    w = wr[slot][l, pl.ds(32 * j, 32)]
                    we, wo = plsc.unpack(
                        w, format=plsc.PackFormat.INTERLEAVED)
                    a[2 * j] = a[2 * j] + we
                    a[2 * j + 1] = a[2 * j + 1] + wo
                tb = lane_splat(tids, i)
                for j in range(2):
                    gi = plsc.load_gather(topicv, [tb + iotas[j]])
                    gb = plsc.bitcast(gi, jnp.bfloat16)
                    te, to = plsc.unpack(
                        gb, format=plsc.PackFormat.INTERLEAVED)
                    a[4 + 2 * j] = a[4 + 2 * j] + te
                    a[4 + 2 * j + 1] = a[4 + 2 * j + 1] + to
                return tuple(a) + (tids,)
            return inner

        def outer(m, acc):
            tids = idxt[r_out, pl.ds(16 * m, 16)] * TOPIC_WORDS
            res = lax.fori_loop(0, 16, lane_body(16 * m), acc[:8] + (tids,),
                                unroll=4)
            return res[:8]

        zeros = tuple(jnp.zeros((16,), jnp.float32) for _ in range(8))
        acc = lax.fori_loop(0, L // 16, outer, zeros)
        tids = idxt[r_out, pl.ds(L - 16, 16)] * TOPIC_WORDS
        acc = lax.fori_loop(16 - L % 16, 16, lane_body(L - 16),
                            acc + (tids,), unroll=4)[:8]
        scale = jnp.float32(1.0 / L)
        for j in range(8):
            outv[r_out, pl.ds(16 * j, 16)] = acc[j] * scale

    def block_body(k, carry):
        row0 = base + k * IB
        pltpu.sync_copy(word_ids_hbm.at[pl.ds(row0, IB)], idxw)
        pltpu.sync_copy(topic_ids_hbm.at[pl.ds(row0, IB)], idxt)
        pending = {r % NSLOT: issue(r, r % NSLOT) for r in range(LOOKAHEAD)}
        for r in range(IB):
            s = r % NSLOT
            for c in pending[s]:
                c.wait()
            nxt = r + LOOKAHEAD
            if nxt < IB:
                pending[nxt % NSLOT] = issue(nxt, nxt % NSLOT)
            reduce_row(s, r)
        pltpu.sync_copy(outv, out_hbm.at[pl.ds(row0, IB)])
        return carry

    lax.fori_loop(0, NBLK, block_body, 0)


def _pooled(word_ids, topic_ids, emb_word_bf16, emb_topics_packed):
    mesh = plsc.VectorSubcoreMesh(core_axis_name="c", subcore_axis_name="s",
                                  num_cores=NC, num_subcores=NS)
    f = pl.kernel(
        _pooled_sc_kernel,
        out_type=jax.ShapeDtypeStruct((B, D), jnp.float32),
        mesh=mesh,
        scratch_types=[
            pltpu.VMEM((IB, L), jnp.int32),
            pltpu.VMEM((IB, L), jnp.int32),
        ] + [pltpu.VMEM((L, EMBED), jnp.bfloat16) for _ in range(NSLOT)]
          + [
            pltpu.VMEM((N_TOPIC_VOCAB * TOPIC_WORDS,), jnp.int32),
            pltpu.VMEM((IB, D), jnp.float32),
        ] + [pltpu.SemaphoreType.DMA for _ in range(NSLOT)],
        compiler_params=pltpu.CompilerParams(use_tc_tiling_on_sc=False,
                                             needs_layout_passes=False),
    )
    return f(word_ids, topic_ids, emb_word_bf16, emb_topics_packed)


def _cast_pair_kernel(a_ref, b_ref, o_ref):
    o_ref[...] = jnp.concatenate(
        [a_ref[...].astype(jnp.bfloat16), b_ref[...].astype(jnp.bfloat16)],
        axis=1)


def _to_bf16_linear(x, br):
    rows, cols = x.shape
    half = rows // 2
    nblk = half // br
    packed = pl.pallas_call(
        _cast_pair_kernel,
        grid=(nblk,),
        in_specs=[pl.BlockSpec((br, cols), lambda i: (i, 0)),
                  pl.BlockSpec((br, cols), lambda i: (i + nblk, 0))],
        out_specs=pl.BlockSpec((br, 2 * cols), lambda i: (i, 0)),
        out_shape=jax.ShapeDtypeStruct((half, 2 * cols), jnp.bfloat16),
    )(x, x)
    return packed.reshape(rows, cols)


def _mlp_kernel(x_ref, w1_ref, b1_ref, w2_ref, b2_ref, o_ref):
    h = jnp.dot(x_ref[...], w1_ref[...], preferred_element_type=jnp.float32)
    h = jnp.maximum(h + b1_ref[...], 0.0)
    o = jnp.dot(h, w2_ref[...], preferred_element_type=jnp.float32)
    o_ref[...] = o + b2_ref[...]


def _mlp(pooled, W1, b1, W2, b2):
    W1p = W1[_PERM, :]
    W2p = jnp.zeros((HIDDEN, 128), jnp.float32).at[:, :NUM_CLASSES].set(W2)
    b2p = jnp.zeros((1, 128), jnp.float32).at[0, :NUM_CLASSES].set(b2)
    out = pl.pallas_call(
        _mlp_kernel,
        out_shape=jax.ShapeDtypeStruct((B, 128), jnp.float32),
    )(pooled, W1p, b1.reshape(1, HIDDEN), W2p, b2p)
    return out[:, :NUM_CLASSES]


@jax.jit
def kernel(word_ids, topic_ids, emb_word, emb_topics, W1, b1, W2, b2):
    word_ids = word_ids.astype(jnp.int32)
    half = N_VOCAB // 2
    word_ids = (word_ids % half) * 2 + word_ids // half
    topic_ids = topic_ids.astype(jnp.int32)
    emb_word_bf16 = _to_bf16_linear(emb_word, 2000)
    emb_topics_packed = lax.bitcast_convert_type(
        emb_topics.astype(jnp.bfloat16).reshape(-1, 2), jnp.int32)
    pooled = _pooled(word_ids, topic_ids, emb_word_bf16, emb_topics_packed)
    return _mlp(pooled, W1, b1, W2, b2)


# speedup vs baseline: 1.0604x; 1.0604x over previous
import functools

import numpy as np
import jax
import jax.numpy as jnp
from jax import lax
from jax.experimental import pallas as pl
from jax.experimental.pallas import tpu as pltpu
from jax.experimental.pallas import tpu_sc as plsc

B, L = 4096, 200
EMBED, TOPICS = 64, 64
D = EMBED + TOPICS
N_VOCAB, N_TOPIC_VOCAB = 100000, 1000
HIDDEN, NUM_CLASSES = 256, 10

NC, NS = 2, 16
NW = NC * NS
B_PER_W = B // NW
IB = 16
NBLK = B_PER_W // IB
NSLOT = 4
LOOKAHEAD = 3
TOPIC_WORDS = TOPICS // 2

CHUNKS = ((0, 128), (128, 72))

_PERM = np.concatenate(
    [np.concatenate([np.arange(g, g + 32, 2), np.arange(g + 1, g + 32, 2)])
     for g in range(0, D, 32)])


def _pooled_sc_kernel(word_ids_hbm, topic_ids_hbm, emb_word_hbm,
                      emb_topics_packed_hbm, out_hbm,
                      idxw, idxt, wr0, wr1, wr2, wr3, topicv,
                      outv, sem0, sem1, sem2, sem3):
    wid = lax.axis_index("s") * NC + lax.axis_index("c")
    base = wid * B_PER_W
    wr = (wr0, wr1, wr2, wr3)
    sems = (sem0, sem1, sem2, sem3)

    pltpu.sync_copy(emb_topics_packed_hbm, topicv)

    iotas = tuple(lax.iota(jnp.int32, 16) + 16 * j for j in range(2))
    _splat_dnums = lax.GatherDimensionNumbers(
        offset_dims=(), collapsed_slice_dims=(0,), start_index_map=(0,))

    def lane_splat(v, i):
        idx = jnp.zeros((16, 1), jnp.int32) + i
        return lax.gather(v, idx, _splat_dnums, (1,),
                          mode=lax.GatherScatterMode.PROMISE_IN_BOUNDS)

    def issue(r, slot):
        copies = []
        for off, n in CHUNKS:
            copies.append(pltpu.async_copy(
                emb_word_hbm.at[idxw.at[r, pl.ds(off, n)]],
                wr[slot].at[pl.ds(off, n)], sems[slot]))
        return copies

    def reduce_row(slot, r_out):
        def lane_body(off):
            def inner(i, acc2):
                l = off + i
                a = list(acc2[:8])
                tids = acc2[8]
                for j in range(2):
                    w = wr[slot][l, pl.ds(32 * j, 32)]
                    we, wo = plsc.unpack(
                        w, format=plsc.PackFormat.INTERLEAVED)
                    a[2 * j] = a[2 * j] + we
                    a[2 * j + 1] = a[2 * j + 1] + wo
                tb = lane_splat(tids, i)
                for j in range(2):
                    gi = plsc.load_gather(topicv, [tb + iotas[j]])
                    gb = plsc.bitcast(gi, jnp.bfloat16)
                    te, to = plsc.unpack(
                        gb, format=plsc.PackFormat.INTERLEAVED)
                    a[4 + 2 * j] = a[4 + 2 * j] + te
                    a[4 + 2 * j + 1] = a[4 + 2 * j + 1] + to
                return tuple(a) + (tids,)
            return inner

        def outer(m, acc):
            tids = idxt[r_out, pl.ds(16 * m, 16)] * TOPIC_WORDS
            res = lax.fori_loop(0, 16, lane_body(16 * m), acc[:8] + (tids,),
                                unroll=8)
            return res[:8]

        zeros = tuple(jnp.zeros((16,), jnp.float32) for _ in range(8))
        acc = lax.fori_loop(0, L // 16, outer, zeros)
        tids = idxt[r_out, pl.ds(L - 16, 16)] * TOPIC_WORDS
        acc = lax.fori_loop(16 - L % 16, 16, lane_body(L - 16),
                            acc + (tids,), unroll=8)[:8]
        scale = jnp.float32(1.0 / L)
        for j in range(8):
            outv[r_out, pl.ds(16 * j, 16)] = acc[j] * scale

    def block_body(k, carry):
        row0 = base + k * IB
        pltpu.sync_copy(word_ids_hbm.at[pl.ds(row0, IB)], idxw)
        pltpu.sync_copy(topic_ids_hbm.at[pl.ds(row0, IB)], idxt)
        pending = {r % NSLOT: issue(r, r % NSLOT) for r in range(LOOKAHEAD)}
        for r in range(IB):
            s = r % NSLOT
            for c in pending[s]:
                c.wait()
            nxt = r + LOOKAHEAD
            if nxt < IB:
                pending[nxt % NSLOT] = issue(nxt, nxt % NSLOT)
            reduce_row(s, r)
        pltpu.sync_copy(outv, out_hbm.at[pl.ds(row0, IB)])
        return carry

    lax.fori_loop(0, NBLK, block_body, 0)


def _pooled(word_ids, topic_ids, emb_word_bf16, emb_topics_packed):
    mesh = plsc.VectorSubcoreMesh(core_axis_name="c", subcore_axis_name="s",
                                  num_cores=NC, num_subcores=NS)
    f = pl.kernel(
        _pooled_sc_kernel,
        out_type=jax.ShapeDtypeStruct((B, D), jnp.float32),
        mesh=mesh,
        scratch_types=[
            pltpu.VMEM((IB, L), jnp.int32),
            pltpu.VMEM((IB, L), jnp.int32),
        ] + [pltpu.VMEM((L, EMBED), jnp.bfloat16) for _ in range(NSLOT)]
          + [
            pltpu.VMEM((N_TOPIC_VOCAB * TOPIC_WORDS,), jnp.int32),
            pltpu.VMEM((IB, D), jnp.float32),
        ] + [pltpu.SemaphoreType.DMA for _ in range(NSLOT)],
        compiler_params=pltpu.CompilerParams(use_tc_tiling_on_sc=False,
                                             needs_layout_passes=False),
    )
    return f(word_ids, topic_ids, emb_word_bf16, emb_topics_packed)


def _mlp_kernel(x_ref, w1_ref, b1_ref, w2_ref, b2_ref, o_ref):
    h = jnp.dot(x_ref[...], w1_ref[...], preferred_element_type=jnp.float32)
    h = jnp.maximum(h + b1_ref[...], 0.0)
    o = jnp.dot(h, w2_ref[...], preferred_element_type=jnp.float32)
    o_ref[...] = o + b2_ref[...]


def _mlp(pooled, W1, b1, W2, b2):
    W1p = W1[_PERM, :]
    W2p = jnp.zeros((HIDDEN, 128), jnp.float32).at[:, :NUM_CLASSES].set(W2)
    b2p = jnp.zeros((1, 128), jnp.float32).at[0, :NUM_CLASSES].set(b2)
    out = pl.pallas_call(
        _mlp_kernel,
        out_shape=jax.ShapeDtypeStruct((B, 128), jnp.float32),
    )(pooled, W1p, b1.reshape(1, HIDDEN), W2p, b2p)
    return out[:, :NUM_CLASSES]


@jax.jit
def kernel(word_ids, topic_ids, emb_word, emb_topics, W1, b1, W2, b2):
    word_ids = word_ids.astype(jnp.int32)
    topic_ids = topic_ids.astype(jnp.int32)
    emb_word_bf16 = emb_word.astype(jnp.bfloat16)
    emb_topics_packed = lax.bitcast_convert_type(
        emb_topics.astype(jnp.bfloat16).reshape(-1, 2), jnp.int32)
    pooled = _pooled(word_ids, topic_ids, emb_word_bf16, emb_topics_packed)
    return _mlp(pooled, W1, b1, W2, b2)


# speedup vs baseline: 1.1324x; 1.0679x over previous
import functools

import numpy as np
import jax
import jax.numpy as jnp
from jax import lax
from jax.experimental import pallas as pl
from jax.experimental.pallas import tpu as pltpu
from jax.experimental.pallas import tpu_sc as plsc

B, L = 4096, 200
EMBED, TOPICS = 64, 64
D = EMBED + TOPICS
N_VOCAB, N_TOPIC_VOCAB = 100000, 1000
HIDDEN, NUM_CLASSES = 256, 10

NC, NS = 2, 16
NW = NC * NS
B_PER_W = B // NW
IB = 16
NBLK = B_PER_W // IB
NSLOT = 4
LOOKAHEAD = 3
TOPIC_WORDS = TOPICS // 2

CHUNKS = ((0, 128), (128, 72))

_PERM = np.concatenate(
    [np.concatenate([np.arange(g, g + 32, 2), np.arange(g + 1, g + 32, 2)])
     for g in range(0, D, 32)])


def _pooled_sc_kernel(word_ids_hbm, topic_ids_hbm, emb_word_hbm,
                      emb_topics_packed_hbm, out_hbm,
                      idxw, idxt, wr0, wr1, wr2, wr3, topicv,
                      outv, sem0, sem1, sem2, sem3):
    wid = lax.axis_index("s") * NC + lax.axis_index("c")
    base = wid * B_PER_W
    wr = (wr0, wr1, wr2, wr3)
    sems = (sem0, sem1, sem2, sem3)

    pltpu.sync_copy(emb_topics_packed_hbm, topicv)

    iotas = tuple(lax.iota(jnp.int32, 16) + 16 * j for j in range(2))
    _splat_dnums = lax.GatherDimensionNumbers(
        offset_dims=(), collapsed_slice_dims=(0,), start_index_map=(0,))

    def lane_splat(v, i):
        idx = jnp.zeros((16, 1), jnp.int32) + i
        return lax.gather(v, idx, _splat_dnums, (1,),
                          mode=lax.GatherScatterMode.PROMISE_IN_BOUNDS)

    def issue(r, slot):
        copies = []
        for off, n in CHUNKS:
            copies.append(pltpu.async_copy(
                emb_word_hbm.at[idxw.at[r, pl.ds(off, n)]],
                wr[slot].at[pl.ds(off, n)], sems[slot]))
        return copies

    def reduce_row(slot, r_out):
        def lane_body(off):
            def inner(i, acc2):
                l = off + i
                a = list(acc2[:8])
                tids = acc2[8]
                for j in range(2):
                    w = wr[slot][l, pl.ds(32 * j, 32)]
                    we, wo = plsc.unpack(
                        w, format=plsc.PackFormat.INTERLEAVED)
                    a[2 * j] = a[2 * j] + we
                    a[2 * j + 1] = a[2 * j + 1] + wo
                tb = lane_splat(tids, i)
                for j in range(2):
                    gi = plsc.load_gather(topicv, [tb + iotas[j]])
                    gb = plsc.bitcast(gi, jnp.bfloat16)
                    te, to = plsc.unpack(
                        gb, format=plsc.PackFormat.INTERLEAVED)
                    a[4 + 2 * j] = a[4 + 2 * j] + te
                    a[4 + 2 * j + 1] = a[4 + 2 * j + 1] + to
                return tuple(a) + (tids,)
            return inner

        def outer(m, acc):
            tids = idxt[r_out, pl.ds(16 * m, 16)] * TOPIC_WORDS
            res = lax.fori_loop(0, 16, lane_body(16 * m), acc[:8] + (tids,),
                                unroll=4)
            return res[:8]

        zeros = tuple(jnp.zeros((16,), jnp.float32) for _ in range(8))
        acc = lax.fori_loop(0, L // 16, outer, zeros)
        tids = idxt[r_out, pl.ds(L - 16, 16)] * TOPIC_WORDS
        acc = lax.fori_loop(16 - L % 16, 16, lane_body(L - 16),
                            acc + (tids,), unroll=4)[:8]
        scale = jnp.float32(1.0 / L)
        for j in range(8):
            outv[r_out, pl.ds(16 * j, 16)] = acc[j] * scale

    def block_body(k, carry):
        row0 = base + k * IB
        pltpu.sync_copy(word_ids_hbm.at[pl.ds(row0, IB)], idxw)
        pltpu.sync_copy(topic_ids_hbm.at[pl.ds(row0, IB)], idxt)
        pending = {r % NSLOT: issue(r, r % NSLOT) for r in range(LOOKAHEAD)}
        for r in range(IB):
            s = r % NSLOT
            for c in pending[s]:
                c.wait()
            nxt = r + LOOKAHEAD
            if nxt < IB:
                pending[nxt % NSLOT] = issue(nxt, nxt % NSLOT)
            reduce_row(s, r)
        pltpu.sync_copy(outv, out_hbm.at[pl.ds(row0, IB)])
        return carry

    lax.fori_loop(0, NBLK, block_body, 0)


def _pooled(word_ids, topic_ids, emb_word_bf16, emb_topics_packed):
    mesh = plsc.VectorSubcoreMesh(core_axis_name="c", subcore_axis_name="s",
                                  num_cores=NC, num_subcores=NS)
    f = pl.kernel(
        _pooled_sc_kernel,
        out_type=jax.ShapeDtypeStruct((B, D), jnp.float32),
        mesh=mesh,
        scratch_types=[
            pltpu.VMEM((IB, L), jnp.int32),
            pltpu.VMEM((IB, L), jnp.int32),
        ] + [pltpu.VMEM((L, EMBED), jnp.bfloat16) for _ in range(NSLOT)]
          + [
            pltpu.VMEM((N_TOPIC_VOCAB * TOPIC_WORDS,), jnp.int32),
            pltpu.VMEM((IB, D), jnp.float32),
        ] + [pltpu.SemaphoreType.DMA for _ in range(NSLOT)],
        compiler_params=pltpu.CompilerParams(use_tc_tiling_on_sc=False,
                                             needs_layout_passes=False),
    )
    return f(word_ids, topic_ids, emb_word_bf16, emb_topics_packed)


def _mlp_kernel(x_ref, w1_ref, b1_ref, w2_ref, b2_ref, o_ref):
    h = jnp.dot(x_ref[...], w1_ref[...], preferred_element_type=jnp.float32)
    h = jnp.maximum(h + b1_ref[...], 0.0)
    o = jnp.dot(h, w2_ref[...], preferred_element_type=jnp.float32)
    o_ref[...] = o + b2_ref[...]


def _mlp(pooled, W1, b1, W2, b2):
    W1p = W1[_PERM, :]
    W2p = jnp.zeros((HIDDEN, 128), jnp.float32).at[:, :NUM_CLASSES].set(W2)
    b2p = jnp.zeros((1, 128), jnp.float32).at[0, :NUM_CLASSES].set(b2)
    out = pl.pallas_call(
        _mlp_kernel,
        out_shape=jax.ShapeDtypeStruct((B, 128), jnp.float32),
    )(pooled, W1p, b1.reshape(1, HIDDEN), W2p, b2p)
    return out[:, :NUM_CLASSES]


@jax.jit
def kernel(word_ids, topic_ids, emb_word, emb_topics, W1, b1, W2, b2):
    word_ids = word_ids.astype(jnp.int32)
    topic_ids = topic_ids.astype(jnp.int32)
    emb_word_bf16 = emb_word.astype(jnp.bfloat16)
    emb_topics_packed = lax.bitcast_convert_type(
        emb_topics.astype(jnp.bfloat16).reshape(-1, 2), jnp.int32)
    pooled = _pooled(word_ids, topic_ids, emb_word_bf16, emb_topics_packed)
    return _mlp(pooled, W1, b1, W2, b2)
